# baseline (device time: 37063 ns/iter reference)
import jax
import jax.numpy as jnp
from jax import lax
from jax.experimental import pallas as pl
from jax.experimental.pallas import tpu as pltpu

N_CHUNKS = 32


def kernel(x):
    m_per, n = x.shape
    n_out = n // 2
    m_glob = 2 * m_per
    m_half = m_per // 2
    c_rows = m_half // N_CHUNKS

    def body(x_ref, out_ref, x_send, x_recv, y_send, y_recv, local_sem):
        mx = lax.axis_index("x")
        my = lax.axis_index("y")
        px = 1 - mx
        py = 1 - my

        barrier_sem = pltpu.get_barrier_semaphore()
        pl.semaphore_signal(
            barrier_sem, inc=1,
            device_id=(px, my), device_id_type=pl.DeviceIdType.MESH,
        )
        pl.semaphore_signal(
            barrier_sem, inc=1,
            device_id=(mx, py), device_id_type=pl.DeviceIdType.MESH,
        )
        pl.semaphore_wait(barrier_sem, 2)

        x_rdmas = []
        for i in range(N_CHUNKS):
            src_row = my * m_half + i * c_rows
            dst_row = mx * m_per + my * m_half + i * c_rows
            r = pltpu.make_async_remote_copy(
                src_ref=x_ref.at[pl.ds(src_row, c_rows), pl.ds(px * n_out, n_out)],
                dst_ref=out_ref.at[pl.ds(dst_row, c_rows), :],
                send_sem=x_send.at[i],
                recv_sem=x_recv.at[i],
                device_id=(px, my),
                device_id_type=pl.DeviceIdType.MESH,
            )
            r.start()
            x_rdmas.append(r)

        local_copy = pltpu.make_async_copy(
            x_ref.at[:, pl.ds(mx * n_out, n_out)],
            out_ref.at[pl.ds(mx * m_per, m_per), :],
            local_sem,
        )
        local_copy.start()

        y_rdmas = []
        for i in range(N_CHUNKS):
            x_rdmas[i].wait_recv()
            rrow = px * m_per + my * m_half + i * c_rows
            r = pltpu.make_async_remote_copy(
                src_ref=out_ref.at[pl.ds(rrow, c_rows), :],
                dst_ref=out_ref.at[pl.ds(rrow, c_rows), :],
                send_sem=y_send.at[i],
                recv_sem=y_recv.at[i],
                device_id=(mx, py),
                device_id_type=pl.DeviceIdType.MESH,
            )
            r.start()
            y_rdmas.append(r)

        for i in range(N_CHUNKS):
            y_rdmas[i].wait_recv()
            x_rdmas[i].wait_send()
            y_rdmas[i].wait_send()
        local_copy.wait()

    return pl.pallas_call(
        body,
        out_shape=jax.ShapeDtypeStruct((m_glob, n_out), x.dtype),
        in_specs=[pl.BlockSpec(memory_space=pltpu.VMEM)],
        out_specs=pl.BlockSpec(memory_space=pltpu.VMEM),
        scratch_shapes=[
            pltpu.SemaphoreType.DMA((N_CHUNKS,)),
            pltpu.SemaphoreType.DMA((N_CHUNKS,)),
            pltpu.SemaphoreType.DMA((N_CHUNKS,)),
            pltpu.SemaphoreType.DMA((N_CHUNKS,)),
            pltpu.SemaphoreType.DMA,
        ],
        compiler_params=pltpu.CompilerParams(collective_id=0),
    )(x)
